# baseline (speedup 1.0000x reference)
import jax
import jax.numpy as jnp
from jax.experimental import pallas as pl

SUB = 200
BM = 2 * SUB
_BN_RSQRT = (1.0 + 1e-5) ** -0.5


def _mlp(pooled, w1_ref, b1_ref, g1_ref, bt1_ref, w2_ref, b2_ref, g2_ref,
         bt2_ref):
    s1 = g1_ref[...] * _BN_RSQRT
    t = jax.lax.dot_general(
        pooled, w1_ref[...], (((1,), (0,)), ((), ())),
        preferred_element_type=jnp.float32)
    t = jnp.maximum(t * s1 + (b1_ref[...] * s1 + bt1_ref[...]), 0.0)
    s2 = g2_ref[...] * _BN_RSQRT
    t = jax.lax.dot_general(
        t, w2_ref[...], (((1,), (0,)), ((), ())),
        preferred_element_type=jnp.float32)
    return jnp.maximum(t * s2 + (b2_ref[...] * s2 + bt2_ref[...]), 0.0)


def _pooled_half(adj_ref, hfull_ref, hblk_ref, eps, lo):
    p = jax.lax.dot_general(
        adj_ref[...], hfull_ref[...], (((1,), (0,)), ((), ())),
        preferred_element_type=jnp.float32)
    return p + (1.0 + eps) * hblk_ref[lo:lo + SUB, :]


def _gin_layer_kernel(adj_a_ref, adj_b_ref, hfull_ref, hblk_ref, eps_ref,
                      w1_ref, b1_ref, g1_ref, bt1_ref,
                      w2_ref, b2_ref, g2_ref, bt2_ref, out_ref):
    eps = eps_ref[0, 0]
    for k, aref in enumerate((adj_a_ref, adj_b_ref)):
        pooled = _pooled_half(aref, hfull_ref, hblk_ref, eps, k * SUB)
        out_ref[k * SUB:(k + 1) * SUB, :] = _mlp(
            pooled, w1_ref, b1_ref, g1_ref, bt1_ref,
            w2_ref, b2_ref, g2_ref, bt2_ref)


def _gin_head_kernel(adj_a_ref, adj_b_ref, hfull_ref, hblk_ref, eps_ref,
                     w1_ref, b1_ref, g1_ref, bt1_ref,
                     w2_ref, b2_ref, g2_ref, bt2_ref,
                     wp_ref, bp_ref, out_ref):
    eps = eps_ref[0, 1]
    for k, aref in enumerate((adj_a_ref, adj_b_ref)):
        pooled = _pooled_half(aref, hfull_ref, hblk_ref, eps, k * SUB)
        h2 = _mlp(pooled, w1_ref, b1_ref, g1_ref, bt1_ref,
                  w2_ref, b2_ref, g2_ref, bt2_ref)
        score = jax.lax.dot_general(
            h2, wp_ref[...], (((1,), (0,)), ((), ())),
            preferred_element_type=jnp.float32)
        out_ref[k * SUB:(k + 1) * SUB, :] = score + bp_ref[0, 0]


def _layer_specs(n, d, head):
    grid = (n // BM,)
    full = lambda i: (0, 0)
    vec = pl.BlockSpec((1, d), full)
    mat = pl.BlockSpec((d, d), full)
    in_specs = [
        pl.BlockSpec((SUB, n), lambda i: (2 * i, 0)),
        pl.BlockSpec((SUB, n), lambda i: (2 * i + 1, 0)),
        pl.BlockSpec((n, d), full),
        pl.BlockSpec((BM, d), lambda i: (i, 0)),
        pl.BlockSpec((1, 2), full),
        mat, vec, vec, vec,
        mat, vec, vec, vec,
    ]
    if head:
        in_specs += [
            pl.BlockSpec((d, 1), full),
            pl.BlockSpec((1, 1), full),
        ]
    return grid, in_specs


@jax.jit
def _run(seq1, adj, eps,
         l0_W1, l0_b1, l0_bn1_g, l0_bn1_b, l0_W2, l0_b2, l0_bn2_g, l0_bn2_b,
         l1_W1, l1_b1, l1_bn1_g, l1_bn1_b, l1_W2, l1_b2, l1_bn2_g, l1_bn2_b,
         Wp, bp):
    n, d = seq1.shape
    r = lambda v: v.reshape(1, d)
    eps2 = eps.reshape(1, 2)

    grid, in_specs = _layer_specs(n, d, head=False)
    h1 = pl.pallas_call(
        _gin_layer_kernel,
        grid=grid,
        in_specs=in_specs,
        out_specs=pl.BlockSpec((BM, d), lambda i: (i, 0)),
        out_shape=jax.ShapeDtypeStruct((n, d), jnp.float32),
    )(adj, adj, seq1, seq1, eps2,
      l0_W1, r(l0_b1), r(l0_bn1_g), r(l0_bn1_b),
      l0_W2, r(l0_b2), r(l0_bn2_g), r(l0_bn2_b))

    grid, in_specs = _layer_specs(n, d, head=True)
    score = pl.pallas_call(
        _gin_head_kernel,
        grid=grid,
        in_specs=in_specs,
        out_specs=pl.BlockSpec((BM, 1), lambda i: (i, 0)),
        out_shape=jax.ShapeDtypeStruct((n, 1), jnp.float32),
    )(adj, adj, h1, h1, eps2,
      l1_W1, r(l1_b1), r(l1_bn1_g), r(l1_bn1_b),
      l1_W2, r(l1_b2), r(l1_bn2_g), r(l1_bn2_b),
      Wp, bp.reshape(1, 1))
    return score


def kernel(seq1, adj, eps,
           l0_W1, l0_b1, l0_bn1_g, l0_bn1_b, l0_W2, l0_b2, l0_bn2_g, l0_bn2_b,
           l1_W1, l1_b1, l1_bn1_g, l1_bn1_b, l1_W2, l1_b2, l1_bn2_g, l1_bn2_b,
           Wp, bp):
    return _run(seq1, adj, eps,
                l0_W1, l0_b1, l0_bn1_g, l0_bn1_b, l0_W2, l0_b2, l0_bn2_g,
                l0_bn2_b, l1_W1, l1_b1, l1_bn1_g, l1_bn1_b, l1_W2, l1_b2,
                l1_bn2_g, l1_bn2_b, Wp, bp)


# speedup vs baseline: 1.1349x; 1.1349x over previous
import jax
import jax.numpy as jnp
from jax.experimental import pallas as pl
from jax.experimental.pallas import tpu as pltpu

BM = 400
_BN_RSQRT = (1.0 + 1e-5) ** -0.5


def _mlp(pooled, w1_ref, b1_ref, g1_ref, bt1_ref, w2_ref, b2_ref, g2_ref,
         bt2_ref):
    s1 = g1_ref[...] * _BN_RSQRT
    t = jax.lax.dot_general(
        pooled, w1_ref[...], (((1,), (0,)), ((), ())),
        preferred_element_type=jnp.float32)
    t = jnp.maximum(t * s1 + (b1_ref[...] * s1 + bt1_ref[...]), 0.0)
    s2 = g2_ref[...] * _BN_RSQRT
    t = jax.lax.dot_general(
        t, w2_ref[...], (((1,), (0,)), ((), ())),
        preferred_element_type=jnp.float32)
    return jnp.maximum(t * s2 + (b2_ref[...] * s2 + bt2_ref[...]), 0.0)


def _make_kernel(nb):
    def body(adj_ref, h0full_ref, h0blk_ref, eps_ref,
             w10_ref, b10_ref, g10_ref, bt10_ref,
             w20_ref, b20_ref, g20_ref, bt20_ref,
             w11_ref, b11_ref, g11_ref, bt11_ref,
             w21_ref, b21_ref, g21_ref, bt21_ref,
             wp_ref, bp_ref, out_ref, h1_ref):
        i = pl.program_id(0)

        @pl.when(i < nb)
        def _layer0():
            pooled = jax.lax.dot_general(
                adj_ref[...], h0full_ref[...], (((1,), (0,)), ((), ())),
                preferred_element_type=jnp.float32)
            pooled = pooled + (1.0 + eps_ref[0, 0]) * h0blk_ref[...]
            h1_ref[pl.ds(i * BM, BM), :] = _mlp(
                pooled, w10_ref, b10_ref, g10_ref, bt10_ref,
                w20_ref, b20_ref, g20_ref, bt20_ref)

        @pl.when(i >= nb)
        def _layer1_head():
            j = i - nb
            pooled = jax.lax.dot_general(
                adj_ref[...], h1_ref[...], (((1,), (0,)), ((), ())),
                preferred_element_type=jnp.float32)
            pooled = pooled + (1.0 + eps_ref[0, 1]) * h1_ref[pl.ds(j * BM, BM), :]
            h2 = _mlp(pooled, w11_ref, b11_ref, g11_ref, bt11_ref,
                      w21_ref, b21_ref, g21_ref, bt21_ref)
            score = jax.lax.dot_general(
                h2, wp_ref[...], (((1,), (0,)), ((), ())),
                preferred_element_type=jnp.float32)
            out_ref[...] = score + bp_ref[0, 0]

    return body


@jax.jit
def _run(seq1, adj, eps,
         l0_W1, l0_b1, l0_bn1_g, l0_bn1_b, l0_W2, l0_b2, l0_bn2_g, l0_bn2_b,
         l1_W1, l1_b1, l1_bn1_g, l1_bn1_b, l1_W2, l1_b2, l1_bn2_g, l1_bn2_b,
         Wp, bp):
    n, d = seq1.shape
    nb = n // BM
    r = lambda v: v.reshape(1, d)
    full = lambda i: (0, 0)
    vec = pl.BlockSpec((1, d), full)
    mat = pl.BlockSpec((d, d), full)
    in_specs = [
        pl.BlockSpec((BM, n), lambda i: (i % nb, 0)),
        pl.BlockSpec((n, d), full),
        pl.BlockSpec((BM, d), lambda i: (i % nb, 0)),
        pl.BlockSpec((1, 2), full),
        mat, vec, vec, vec,
        mat, vec, vec, vec,
        mat, vec, vec, vec,
        mat, vec, vec, vec,
        pl.BlockSpec((d, 1), full),
        pl.BlockSpec((1, 1), full),
    ]
    score = pl.pallas_call(
        _make_kernel(nb),
        grid=(2 * nb,),
        in_specs=in_specs,
        out_specs=pl.BlockSpec(
            (BM, 1), lambda i: (jnp.where(i < nb, 0, i - nb), 0)),
        out_shape=jax.ShapeDtypeStruct((n, 1), jnp.float32),
        scratch_shapes=[pltpu.VMEM((n, d), jnp.float32)],
    )(adj, seq1, seq1, eps.reshape(1, 2),
      l0_W1, r(l0_b1), r(l0_bn1_g), r(l0_bn1_b),
      l0_W2, r(l0_b2), r(l0_bn2_g), r(l0_bn2_b),
      l1_W1, r(l1_b1), r(l1_bn1_g), r(l1_bn1_b),
      l1_W2, r(l1_b2), r(l1_bn2_g), r(l1_bn2_b),
      Wp, bp.reshape(1, 1))
    return score


def kernel(seq1, adj, eps,
           l0_W1, l0_b1, l0_bn1_g, l0_bn1_b, l0_W2, l0_b2, l0_bn2_g, l0_bn2_b,
           l1_W1, l1_b1, l1_bn1_g, l1_bn1_b, l1_W2, l1_b2, l1_bn2_g, l1_bn2_b,
           Wp, bp):
    return _run(seq1, adj, eps,
                l0_W1, l0_b1, l0_bn1_g, l0_bn1_b, l0_W2, l0_b2, l0_bn2_g,
                l0_bn2_b, l1_W1, l1_b1, l1_bn1_g, l1_bn1_b, l1_W2, l1_b2,
                l1_bn2_g, l1_bn2_b, Wp, bp)
